# baseline (speedup 1.0000x reference)
import functools

import jax
import jax.numpy as jnp
from jax import lax
from jax.experimental import pallas as pl
from jax.experimental.pallas import tpu as pltpu
from jax.experimental.pallas import tpu_sc as plsc

_CHUNK = 32
_NBUF = 2
_ROWS = 512


def _sc_lookup_last_slot(weights, batch):
    seq_len, emb = weights.shape
    info = plsc.get_sparse_core_info()
    num_workers = info.num_cores * info.num_subcores
    rows_per_w = seq_len // num_workers
    n_chunks = rows_per_w // _CHUNK

    mesh = plsc.VectorSubcoreMesh(core_axis_name="c", subcore_axis_name="s")

    @functools.partial(
        pl.kernel,
        out_type=jax.ShapeDtypeStruct((batch, seq_len, emb), weights.dtype),
        mesh=mesh,
        scratch_types=[
            pltpu.VMEM((_NBUF, _CHUNK, emb), jnp.float32),
            pltpu.SemaphoreType.DMA,
            pltpu.SemaphoreType.DMA,
        ],
    )
    def _lookup(w_hbm, out_hbm, buf, gsem, ssem):
        wid = lax.axis_index("s") * info.num_cores + lax.axis_index("c")
        base = wid * rows_per_w

        def gather(ci):
            return pltpu.make_async_copy(
                w_hbm.at[pl.ds(base + ci * _CHUNK, _CHUNK)],
                buf.at[ci % _NBUF],
                gsem,
            )

        def scatter(ci):
            return pltpu.make_async_copy(
                buf.at[ci % _NBUF],
                out_hbm.at[batch - 1, pl.ds(base + ci * _CHUNK, _CHUNK)],
                ssem,
            )

        gather(0).start()
        for ci in range(n_chunks):
            if ci + 1 < n_chunks:
                if ci + 1 >= _NBUF:
                    scatter(ci + 1 - _NBUF).wait()
                gather(ci + 1).start()
            gather(ci).wait()
            scatter(ci).start()
        for ci in range(max(0, n_chunks - _NBUF), n_chunks):
            scatter(ci).wait()

    return _lookup(weights)


def _tc_body(w_ref, _a_ref, o_ref):
    o_ref[...] = w_ref[...][None]


def _tc_bcast_rest(weights, out_buf, batch):
    seq_len, emb = weights.shape
    n_blocks = seq_len // _ROWS
    return pl.pallas_call(
        _tc_body,
        grid=(n_blocks, batch - 1),
        in_specs=[
            pl.BlockSpec((_ROWS, emb), lambda i, b: (i, 0)),
            pl.BlockSpec(memory_space=pl.ANY),
        ],
        out_specs=pl.BlockSpec((1, _ROWS, emb), lambda i, b: (b, i, 0)),
        out_shape=jax.ShapeDtypeStruct((batch, seq_len, emb), weights.dtype),
        input_output_aliases={1: 0},
    )(weights, out_buf)


def kernel(input_seq, weights):
    batch, _ = input_seq.shape
    out_buf = _sc_lookup_last_slot(weights, batch)
    return _tc_bcast_rest(weights, out_buf, batch)


# speedup vs baseline: 1.3077x; 1.3077x over previous
import functools

import jax
import jax.numpy as jnp
from jax import lax
from jax.experimental import pallas as pl
from jax.experimental.pallas import tpu as pltpu
from jax.experimental.pallas import tpu_sc as plsc

_CHUNK = 16
_NBUF = 3


def kernel(input_seq, weights):
    batch, seq_len = input_seq.shape
    _, emb = weights.shape

    info = plsc.get_sparse_core_info()
    num_workers = info.num_cores * info.num_subcores
    rows_per_w = seq_len // num_workers
    n_chunks = rows_per_w // _CHUNK
    depth = 2 * _NBUF

    mesh = plsc.VectorSubcoreMesh(core_axis_name="c", subcore_axis_name="s")

    @functools.partial(
        pl.kernel,
        out_type=jax.ShapeDtypeStruct((batch, seq_len, emb), weights.dtype),
        mesh=mesh,
        scratch_types=[
            pltpu.VMEM((_NBUF, _CHUNK, emb), jnp.float32),
            pltpu.VMEM_SHARED((info.num_subcores, _NBUF, _CHUNK, emb), jnp.float32),
            pltpu.SemaphoreType.DMA,
            pltpu.SemaphoreType.DMA,
            pltpu.SemaphoreType.DMA,
            pltpu.SemaphoreType.DMA,
        ],
    )
    def _bcast(w_hbm, out_hbm, tbuf, sbuf, tg, ts, sg, ss):
        cid = lax.axis_index("c")
        sid = lax.axis_index("s")
        wid = sid * info.num_cores + cid
        base = wid * rows_per_w

        def buf(ci):
            j = (ci // 2) % _NBUF
            if ci % 2 == 0:
                return tbuf.at[j]
            return sbuf.at[sid, j]

        def gather(ci):
            return pltpu.make_async_copy(
                w_hbm.at[pl.ds(base + ci * _CHUNK, _CHUNK)],
                buf(ci),
                tg if ci % 2 == 0 else sg,
            )

        def scatters(ci):
            return [
                pltpu.make_async_copy(
                    buf(ci),
                    out_hbm.at[b, pl.ds(base + ci * _CHUNK, _CHUNK)],
                    ts if ci % 2 == 0 else ss,
                )
                for b in range(batch)
            ]

        for ci in range(min(depth, n_chunks)):
            gather(ci).start()
        drained = 0
        for ci in range(n_chunks):
            gather(ci).wait()
            for c in scatters(ci):
                c.start()
            j = ci - 1
            if j >= 0 and j + depth < n_chunks:
                for c in scatters(j):
                    c.wait()
                drained = j + 1
                gather(j + depth).start()
        for ci in range(drained, n_chunks):
            for c in scatters(ci):
                c.wait()

    return _bcast(weights)


# speedup vs baseline: 1.4061x; 1.0752x over previous
import functools

import jax
import jax.numpy as jnp
from jax import lax
from jax.experimental import pallas as pl
from jax.experimental.pallas import tpu as pltpu
from jax.experimental.pallas import tpu_sc as plsc

_CHUNK = 32
_NBUF = 2


def kernel(input_seq, weights):
    batch, seq_len = input_seq.shape
    _, emb = weights.shape

    info = plsc.get_sparse_core_info()
    num_workers = info.num_cores * info.num_subcores
    rows_per_w = seq_len // num_workers
    n_chunks = rows_per_w // _CHUNK

    mesh = plsc.VectorSubcoreMesh(core_axis_name="c", subcore_axis_name="s")

    @functools.partial(
        pl.kernel,
        out_type=jax.ShapeDtypeStruct((batch, seq_len, emb), weights.dtype),
        mesh=mesh,
        scratch_types=[
            pltpu.VMEM((_NBUF, _CHUNK, emb), jnp.float32),
            pltpu.SemaphoreType.DMA,
            pltpu.SemaphoreType.DMA,
        ],
    )
    def _bcast(w_hbm, out_hbm, buf, gsem, ssem):
        wid = lax.axis_index("s") * info.num_cores + lax.axis_index("c")
        base = wid * rows_per_w

        def gather(ci):
            return pltpu.make_async_copy(
                w_hbm.at[pl.ds(base + ci * _CHUNK, _CHUNK)],
                buf.at[ci % _NBUF],
                gsem,
            )

        def scatters(ci):
            return [
                pltpu.make_async_copy(
                    buf.at[ci % _NBUF],
                    out_hbm.at[b, pl.ds(base + ci * _CHUNK, _CHUNK)],
                    ssem,
                )
                for b in range(batch)
            ]

        gather(0).start()
        for ci in range(n_chunks):
            if ci + 1 < n_chunks:
                if ci + 1 >= _NBUF:
                    for c in scatters(ci + 1 - _NBUF):
                        c.wait()
                gather(ci + 1).start()
            gather(ci).wait()
            for c in scatters(ci):
                c.start()
        for ci in range(max(0, n_chunks - _NBUF), n_chunks):
            for c in scatters(ci):
                c.wait()

    return _bcast(weights)
